# baseline (speedup 1.0000x reference)
import functools

import jax
import jax.numpy as jnp
import numpy as np
from jax import lax
from jax.experimental import pallas as pl
from jax.experimental.pallas import tpu as pltpu
from jax.experimental.pallas import tpu_sc as plsc

_L = 16


def _dataset_params_sc(n_rows, batch):
    nw = 32
    bpw = batch // nw
    ch = bpw // _L
    opw = bpw * 3

    mesh = plsc.VectorSubcoreMesh(core_axis_name="c", subcore_axis_name="s")

    @functools.partial(
        pl.kernel,
        mesh=mesh,
        out_type=(
            jax.ShapeDtypeStruct((batch * 3,), jnp.float32),
            jax.ShapeDtypeStruct((batch,), jnp.float32),
        ),
        scratch_types=[
            pltpu.VMEM((bpw,), jnp.int32),
            pltpu.VMEM((bpw,), jnp.int32),
            pltpu.VMEM((opw,), jnp.int32),
            pltpu.VMEM((opw,), jnp.int32),
            pltpu.VMEM((opw,), jnp.int32),
            pltpu.VMEM((opw,), jnp.float32),
            pltpu.VMEM((opw,), jnp.float32),
            pltpu.VMEM((opw,), jnp.float32),
            pltpu.VMEM((bpw,), jnp.float32),
            pltpu.VMEM((3 * _L,), jnp.int32),
            pltpu.VMEM((3 * _L,), jnp.int32),
            pltpu.VMEM((3 * _L,), jnp.float32),
            pltpu.VMEM((3 * _L,), jnp.float32),
            pltpu.SemaphoreType.DMA,
            pltpu.SemaphoreType.DMA,
            pltpu.SemaphoreType.DMA,
        ],
    )
    def k(ind_hbm, trans_hbm, scale_hbm, gi_hbm, is1_hbm, nzi_hbm, nzf_hbm,
          sm2_hbm, out_t_hbm, out_s_hbm,
          ind_v, idxs_v, gi_v, eind_v, gt_v, sm_v, tv_v, out_t_v, s_v,
          is1_v, nzi_v, nzf_v, sm2_v,
          sem0, sem1, sem2):
        wid = lax.axis_index("s") * 2 + lax.axis_index("c")
        base = wid * bpw

        pltpu.sync_copy(ind_hbm.at[pl.ds(base, bpw)], ind_v)
        pltpu.sync_copy(gi_hbm.at[pl.ds(base * 3, opw)], gi_v)
        pltpu.sync_copy(is1_hbm, is1_v)
        pltpu.sync_copy(nzi_hbm, nzi_v)
        pltpu.sync_copy(nzf_hbm, nzf_v)
        pltpu.sync_copy(sm2_hbm, sm2_v)

        c0 = pltpu.async_copy(ind_hbm.at[gi_v], eind_v, sem0)

        def prep(i, carry):
            sl = pl.ds(i * _L, _L)
            ind = ind_v[sl]
            wrap01 = lax.shift_right_arithmetic(ind - n_rows, 31) + 1
            idxs_v[sl] = ind - n_rows * wrap01
            return carry

        lax.fori_loop(0, ch, prep, 0)
        c2 = pltpu.async_copy(scale_hbm.at[idxs_v], s_v, sem2)
        c0.wait()

        def build(i, carry):
            for m in range(3):
                sl = pl.ds((i * 3 + m) * _L, _L)
                pm = pl.ds(m * _L, _L)
                e = eind_v[sl]
                wrap01 = lax.shift_right_arithmetic(e - n_rows, 31) + 1
                eidx = e - n_rows * wrap01
                gt_v[sl] = (eidx * 2 + is1_v[pm]) * nzi_v[pm]
                sm_v[sl] = nzf_v[pm] * (
                    jnp.float32(1.0)
                    - sm2_v[pm] * wrap01.astype(jnp.float32))
            return carry

        lax.fori_loop(0, ch, build, 0)

        pltpu.async_copy(trans_hbm.at[gt_v], tv_v, sem1).wait()

        def finish(i, carry):
            sl = pl.ds(i * _L, _L)
            out_t_v[sl] = tv_v[sl] * sm_v[sl]
            return carry

        lax.fori_loop(0, 3 * ch, finish, 0)

        pltpu.sync_copy(out_t_v, out_t_hbm.at[pl.ds(base * 3, opw)])
        c2.wait()
        pltpu.sync_copy(s_v, out_s_hbm.at[pl.ds(base, bpw)])

    return k


def kernel(indices, ds_translation, ds_scale):
    n_rows = ds_translation.shape[0]
    batch = indices.shape[0]

    gi_full = jnp.asarray(np.arange(batch * 3) // 3, jnp.int32)
    comp = np.arange(3 * _L) % 3
    is1 = jnp.asarray((comp == 1).astype(np.int32))
    nzi = jnp.asarray((comp != 2).astype(np.int32))
    nzf = jnp.asarray((comp != 2).astype(np.float32))
    sm2 = jnp.asarray(2.0 * (comp == 0).astype(np.float32))

    k = _dataset_params_sc(n_rows, batch)
    out_t, out_s = k(
        indices.astype(jnp.int32),
        ds_translation.reshape(-1),
        ds_scale.reshape(-1),
        gi_full, is1, nzi, nzf, sm2,
    )
    return (out_t.reshape(batch, 3), out_s.reshape(batch, 1))


# speedup vs baseline: 10.9297x; 10.9297x over previous
import functools

import jax
import jax.numpy as jnp
import numpy as np
from jax import lax
from jax.experimental import pallas as pl
from jax.experimental.pallas import tpu as pltpu
from jax.experimental.pallas import tpu_sc as plsc

_L = 16


def _dataset_params_sc(n_rows, batch):
    nw = 32
    bpw = batch // nw
    ch = bpw // _L
    opw = bpw * 3

    mesh = plsc.VectorSubcoreMesh(core_axis_name="c", subcore_axis_name="s")

    @functools.partial(
        pl.kernel,
        mesh=mesh,
        out_type=(
            jax.ShapeDtypeStruct((batch * 3,), jnp.float32),
            jax.ShapeDtypeStruct((batch,), jnp.float32),
        ),
        scratch_types=[
            pltpu.VMEM((bpw,), jnp.int32),
            pltpu.VMEM((bpw,), jnp.int32),
            pltpu.VMEM((bpw,), jnp.float32),
            pltpu.VMEM((bpw,), jnp.float32),
            pltpu.VMEM((bpw,), jnp.float32),
            pltpu.VMEM((bpw,), jnp.float32),
            pltpu.VMEM((opw,), jnp.float32),
            pltpu.VMEM((3 * _L,), jnp.int32),
            pltpu.VMEM((3 * _L,), jnp.int32),
            pltpu.VMEM((3 * _L,), jnp.int32),
            pltpu.SemaphoreType.DMA,
            pltpu.SemaphoreType.DMA,
            pltpu.SemaphoreType.DMA,
        ],
        compiler_params=pltpu.CompilerParams(needs_layout_passes=False),
    )
    def k(ind_hbm, c0_hbm, c1_hbm, sc_hbm, div3_hbm, is0_hbm, is1_hbm,
          out_t_hbm, out_s_hbm,
          ind_v, idx_v, sign_v, t0_v, t1_v, s_v, out_t_v,
          div3_v, is0_v, is1_v,
          sem0, sem1, sem2):
        wid = lax.axis_index("s") * 2 + lax.axis_index("c")
        base = wid * bpw

        pltpu.sync_copy(ind_hbm.at[pl.ds(base, bpw)], ind_v)
        pltpu.sync_copy(div3_hbm, div3_v)
        pltpu.sync_copy(is0_hbm, is0_v)
        pltpu.sync_copy(is1_hbm, is1_v)

        def stage(j, carry):
            sl = pl.ds(j * _L, _L)
            v = ind_v[sl]
            w = v >= n_rows
            idx_v[sl] = jnp.where(w, v - n_rows, v)
            sign_v[sl] = jnp.where(w, jnp.float32(-1.0), jnp.float32(1.0))
            return carry

        lax.fori_loop(0, ch, stage, 0)

        c0 = pltpu.async_copy(c0_hbm.at[idx_v], t0_v, sem0)
        c1 = pltpu.async_copy(c1_hbm.at[idx_v], t1_v, sem1)
        c2 = pltpu.async_copy(sc_hbm.at[idx_v], s_v, sem2)
        c0.wait()
        c1.wait()

        def assemble(i, carry):
            for m in range(3):
                pm = pl.ds(m * _L, _L)
                rows = div3_v[pm] + i * _L
                g0 = plsc.load_gather(t0_v, [rows])
                g1 = plsc.load_gather(t1_v, [rows])
                sg = plsc.load_gather(sign_v, [rows])
                val = jnp.where(
                    is0_v[pm] == 1, g0 * sg,
                    jnp.where(is1_v[pm] == 1, g1, jnp.float32(0.0)))
                out_t_v[pl.ds((i * 3 + m) * _L, _L)] = val
            return carry

        lax.fori_loop(0, ch, assemble, 0)

        pltpu.sync_copy(out_t_v, out_t_hbm.at[pl.ds(base * 3, opw)])
        c2.wait()
        pltpu.sync_copy(s_v, out_s_hbm.at[pl.ds(base, bpw)])

    return k


def kernel(indices, ds_translation, ds_scale):
    n_rows = ds_translation.shape[0]
    batch = indices.shape[0]

    c0 = ds_translation[:, 0]
    c1 = ds_translation[:, 1]
    sc = ds_scale[:, 0]

    p = np.arange(3 * _L)
    comp = p % 3
    div3 = jnp.asarray(p // 3, jnp.int32)
    is0 = jnp.asarray((comp == 0).astype(np.int32))
    is1 = jnp.asarray((comp == 1).astype(np.int32))

    k = _dataset_params_sc(n_rows, batch)
    out_t, out_s = k(
        indices.astype(jnp.int32),
        c0, c1, sc,
        div3, is0, is1,
    )
    return (out_t.reshape(batch, 3), out_s.reshape(batch, 1))


# speedup vs baseline: 12.8386x; 1.1747x over previous
import functools

import jax
import jax.numpy as jnp
from jax import lax
from jax.experimental import pallas as pl
from jax.experimental.pallas import tpu as pltpu
from jax.experimental.pallas import tpu_sc as plsc

_L = 16


def _dataset_params_sc(n_rows, batch):
    nw = 32
    bpw = batch // nw
    ch = bpw // _L

    mesh = plsc.VectorSubcoreMesh(core_axis_name="c", subcore_axis_name="s")

    @functools.partial(
        pl.kernel,
        mesh=mesh,
        out_type=(
            jax.ShapeDtypeStruct((batch,), jnp.float32),
            jax.ShapeDtypeStruct((batch,), jnp.float32),
            jax.ShapeDtypeStruct((batch,), jnp.float32),
        ),
        scratch_types=[
            pltpu.VMEM((bpw,), jnp.int32),
            pltpu.VMEM((bpw,), jnp.int32),
            pltpu.VMEM((bpw,), jnp.float32),
            pltpu.VMEM((bpw,), jnp.float32),
            pltpu.VMEM((bpw,), jnp.float32),
            pltpu.VMEM((bpw,), jnp.float32),
            pltpu.SemaphoreType.DMA,
            pltpu.SemaphoreType.DMA,
            pltpu.SemaphoreType.DMA,
        ],
    )
    def k(ind_hbm, c0_hbm, c1_hbm, sc_hbm, out0_hbm, out1_hbm, outs_hbm,
          ind_v, idx_v, sign_v, t0_v, t1_v, s_v,
          sem0, sem1, sem2):
        wid = lax.axis_index("s") * 2 + lax.axis_index("c")
        base = wid * bpw

        pltpu.sync_copy(ind_hbm.at[pl.ds(base, bpw)], ind_v)

        def stage(j, carry):
            sl = pl.ds(j * _L, _L)
            v = ind_v[sl]
            w = v >= n_rows
            idx_v[sl] = jnp.where(w, v - n_rows, v)
            sign_v[sl] = jnp.where(w, jnp.float32(-1.0), jnp.float32(1.0))
            return carry

        lax.fori_loop(0, ch, stage, 0)

        c0 = pltpu.async_copy(c0_hbm.at[idx_v], t0_v, sem0)
        c1 = pltpu.async_copy(c1_hbm.at[idx_v], t1_v, sem1)
        c2 = pltpu.async_copy(sc_hbm.at[idx_v], s_v, sem2)
        c0.wait()

        def smul(j, carry):
            sl = pl.ds(j * _L, _L)
            t0_v[sl] = t0_v[sl] * sign_v[sl]
            return carry

        lax.fori_loop(0, ch, smul, 0)

        pltpu.sync_copy(t0_v, out0_hbm.at[pl.ds(base, bpw)])
        c1.wait()
        pltpu.sync_copy(t1_v, out1_hbm.at[pl.ds(base, bpw)])
        c2.wait()
        pltpu.sync_copy(s_v, outs_hbm.at[pl.ds(base, bpw)])

    return k


def kernel(indices, ds_translation, ds_scale):
    n_rows = ds_translation.shape[0]
    batch = indices.shape[0]

    c0 = ds_translation[:, 0]
    c1 = ds_translation[:, 1]
    sc = ds_scale[:, 0]

    k = _dataset_params_sc(n_rows, batch)
    t0s, t1, s = k(indices.astype(jnp.int32), c0, c1, sc)

    translation_delta = jnp.stack(
        [t0s, t1, jnp.zeros_like(t0s)], axis=1)
    return (translation_delta, s.reshape(batch, 1))


# speedup vs baseline: 28.9443x; 2.2545x over previous
import functools

import jax
import jax.numpy as jnp
import numpy as np
from jax import lax
from jax.experimental import pallas as pl
from jax.experimental.pallas import tpu as pltpu
from jax.experimental.pallas import tpu_sc as plsc

_L = 16
_R = 128
_CHK = 128


def _dataset_params_sc(n_rows, batch):
    nw = 32
    bpw = batch // nw
    ch = bpw // _L
    nblk = bpw // _CHK

    mesh = plsc.VectorSubcoreMesh(core_axis_name="c", subcore_axis_name="s")

    @functools.partial(
        pl.kernel,
        mesh=mesh,
        out_type=(
            jax.ShapeDtypeStruct((batch,), jnp.float32),
            jax.ShapeDtypeStruct((batch,), jnp.float32),
            jax.ShapeDtypeStruct((batch,), jnp.float32),
        ),
        scratch_types=[
            pltpu.VMEM((bpw,), jnp.int32),
            pltpu.VMEM((bpw,), jnp.int32),
            pltpu.VMEM((bpw,), jnp.int32),
            pltpu.VMEM((bpw,), jnp.float32),
            pltpu.VMEM((_CHK, 2, _R), jnp.float32),
            pltpu.VMEM((_CHK, 1, _R), jnp.float32),
            pltpu.VMEM((bpw,), jnp.float32),
            pltpu.VMEM((bpw,), jnp.float32),
            pltpu.VMEM((bpw,), jnp.float32),
            pltpu.VMEM((_L,), jnp.int32),
            pltpu.SemaphoreType.DMA,
            pltpu.SemaphoreType.DMA,
        ],
        compiler_params=pltpu.CompilerParams(needs_layout_passes=False),
    )
    def k(ind_hbm, xt_hbm, xs_hbm, lin_hbm, out0_hbm, out1_hbm, outs_hbm,
          ind_v, tid_v, lane_v, sign_v, bt_v, bs_v, o0_v, o1_v, os_v, lin_v,
          semt, sems):
        wid = lax.axis_index("s") * 2 + lax.axis_index("c")
        base = wid * bpw

        pltpu.sync_copy(ind_hbm.at[pl.ds(base, bpw)], ind_v)
        pltpu.sync_copy(lin_hbm, lin_v)

        def stage(j, carry):
            sl = pl.ds(j * _L, _L)
            v = ind_v[sl]
            w = v >= n_rows
            idx = jnp.where(w, v - n_rows, v)
            tid_v[sl] = lax.shift_right_logical(idx, 7)
            lane_v[sl] = idx & (_R - 1)
            sign_v[sl] = jnp.where(w, jnp.float32(-1.0), jnp.float32(1.0))
            return carry

        lax.fori_loop(0, ch, stage, 0)

        for blk in range(nblk):
            boff = blk * _CHK
            tslice = tid_v.at[pl.ds(boff, _CHK)]
            ct = pltpu.async_copy(xt_hbm.at[tslice], bt_v, semt)
            cs = pltpu.async_copy(xs_hbm.at[tslice], bs_v, sems)
            ct.wait()
            cs.wait()

            def extract(j, carry):
                gsl = pl.ds(boff + j * _L, _L)
                rows = lin_v[...] + j * _L
                zeros = lin_v[...] * 0
                lanes = lane_v[gsl]
                t0 = plsc.load_gather(bt_v, [rows, zeros, lanes])
                t1 = plsc.load_gather(bt_v, [rows, zeros + 1, lanes])
                sv = plsc.load_gather(bs_v, [rows, zeros, lanes])
                o0_v[gsl] = t0 * sign_v[gsl]
                o1_v[gsl] = t1
                os_v[gsl] = sv
                return carry

            lax.fori_loop(0, _CHK // _L, extract, 0)

        pltpu.sync_copy(o0_v, out0_hbm.at[pl.ds(base, bpw)])
        pltpu.sync_copy(o1_v, out1_hbm.at[pl.ds(base, bpw)])
        pltpu.sync_copy(os_v, outs_hbm.at[pl.ds(base, bpw)])

    return k


def kernel(indices, ds_translation, ds_scale):
    n_rows = ds_translation.shape[0]
    batch = indices.shape[0]
    n_tiles = -(-n_rows // _R)
    pad_rows = n_tiles * _R - n_rows

    xt = jnp.pad(ds_translation, ((0, pad_rows), (0, 0))) \
        .reshape(n_tiles, _R, 2).transpose(0, 2, 1)
    xs = jnp.pad(ds_scale, ((0, pad_rows), (0, 0))) \
        .reshape(n_tiles, _R, 1).transpose(0, 2, 1)
    lin = jnp.asarray(np.arange(_L), jnp.int32)

    k = _dataset_params_sc(n_rows, batch)
    t0s, t1, s = k(indices.astype(jnp.int32), xt, xs, lin)

    translation_delta = jnp.stack([t0s, t1, jnp.zeros_like(t0s)], axis=1)
    return (translation_delta, s.reshape(batch, 1))


# speedup vs baseline: 29.6476x; 1.0243x over previous
import functools

import jax
import jax.numpy as jnp
import numpy as np
from jax import lax
from jax.experimental import pallas as pl
from jax.experimental.pallas import tpu as pltpu
from jax.experimental.pallas import tpu_sc as plsc

_L = 16
_R = 128
_CHK = 128


def _dataset_params_sc(n_rows, batch):
    nw = 32
    bpw = batch // nw
    ch = bpw // _L
    nblk = bpw // _CHK

    mesh = plsc.VectorSubcoreMesh(core_axis_name="c", subcore_axis_name="s")

    @functools.partial(
        pl.kernel,
        mesh=mesh,
        out_type=(
            jax.ShapeDtypeStruct((batch,), jnp.float32),
            jax.ShapeDtypeStruct((batch,), jnp.float32),
            jax.ShapeDtypeStruct((batch,), jnp.float32),
        ),
        scratch_types=[
            pltpu.VMEM((bpw,), jnp.int32),
            pltpu.VMEM((bpw,), jnp.int32),
            pltpu.VMEM((bpw,), jnp.int32),
            pltpu.VMEM((bpw,), jnp.float32),
            pltpu.VMEM((_CHK, 2, _R), jnp.float32),
            pltpu.VMEM((_CHK, 1, _R), jnp.float32),
            pltpu.VMEM((_CHK, 2, _R), jnp.float32),
            pltpu.VMEM((_CHK, 1, _R), jnp.float32),
            pltpu.VMEM((bpw,), jnp.float32),
            pltpu.VMEM((bpw,), jnp.float32),
            pltpu.VMEM((bpw,), jnp.float32),
            pltpu.VMEM((_L,), jnp.int32),
            pltpu.SemaphoreType.DMA,
            pltpu.SemaphoreType.DMA,
            pltpu.SemaphoreType.DMA,
            pltpu.SemaphoreType.DMA,
        ],
        compiler_params=pltpu.CompilerParams(needs_layout_passes=False),
    )
    def k(ind_hbm, xt_hbm, xs_hbm, lin_hbm, out0_hbm, out1_hbm, outs_hbm,
          ind_v, tid_v, lane_v, sign_v, bta_v, bsa_v, btb_v, bsb_v,
          o0_v, o1_v, os_v, lin_v,
          semta, semsa, semtb, semsb):
        wid = lax.axis_index("s") * 2 + lax.axis_index("c")
        base = wid * bpw

        pltpu.sync_copy(ind_hbm.at[pl.ds(base, bpw)], ind_v)
        pltpu.sync_copy(lin_hbm, lin_v)

        def stage(j, carry):
            sl = pl.ds(j * _L, _L)
            v = ind_v[sl]
            w = v >= n_rows
            idx = jnp.where(w, v - n_rows, v)
            tid_v[sl] = lax.shift_right_logical(idx, 7)
            lane_v[sl] = idx & (_R - 1)
            sign_v[sl] = jnp.where(w, jnp.float32(-1.0), jnp.float32(1.0))
            return carry

        lax.fori_loop(0, ch, stage, 0)

        bufs = [(bta_v, bsa_v, semta, semsa), (btb_v, bsb_v, semtb, semsb)]

        def issue(blk):
            bt, bs, st, ss = bufs[blk % 2]
            tslice = tid_v.at[pl.ds(blk * _CHK, _CHK)]
            return (pltpu.async_copy(xt_hbm.at[tslice], bt, st),
                    pltpu.async_copy(xs_hbm.at[tslice], bs, ss))

        pending = {0: issue(0)}
        for blk in range(nblk):
            if blk + 1 < nblk:
                pending[blk + 1] = issue(blk + 1)
            ct, cs = pending.pop(blk)
            ct.wait()
            cs.wait()
            bt_v, bs_v, _, _ = bufs[blk % 2]
            boff = blk * _CHK

            def extract(j, carry, bt_v=bt_v, bs_v=bs_v, boff=boff):
                gsl = pl.ds(boff + j * _L, _L)
                rows = lin_v[...] + j * _L
                zeros = lin_v[...] * 0
                lanes = lane_v[gsl]
                t0 = plsc.load_gather(bt_v, [rows, zeros, lanes])
                t1 = plsc.load_gather(bt_v, [rows, zeros + 1, lanes])
                sv = plsc.load_gather(bs_v, [rows, zeros, lanes])
                o0_v[gsl] = t0 * sign_v[gsl]
                o1_v[gsl] = t1
                os_v[gsl] = sv
                return carry

            lax.fori_loop(0, _CHK // _L, extract, 0)

        pltpu.sync_copy(o0_v, out0_hbm.at[pl.ds(base, bpw)])
        pltpu.sync_copy(o1_v, out1_hbm.at[pl.ds(base, bpw)])
        pltpu.sync_copy(os_v, outs_hbm.at[pl.ds(base, bpw)])

    return k


def kernel(indices, ds_translation, ds_scale):
    n_rows = ds_translation.shape[0]
    batch = indices.shape[0]
    n_tiles = -(-n_rows // _R)
    pad_rows = n_tiles * _R - n_rows

    xt = jnp.pad(ds_translation, ((0, pad_rows), (0, 0))) \
        .reshape(n_tiles, _R, 2).transpose(0, 2, 1)
    xs = jnp.pad(ds_scale, ((0, pad_rows), (0, 0))) \
        .reshape(n_tiles, _R, 1).transpose(0, 2, 1)
    lin = jnp.asarray(np.arange(_L), jnp.int32)

    k = _dataset_params_sc(n_rows, batch)
    t0s, t1, s = k(indices.astype(jnp.int32), xt, xs, lin)

    translation_delta = jnp.stack([t0s, t1, jnp.zeros_like(t0s)], axis=1)
    return (translation_delta, s.reshape(batch, 1))
